# Initial kernel scaffold; baseline (speedup 1.0000x reference)
import functools

import jax
import jax.numpy as jnp
from jax import lax
from jax.experimental import pallas as pl

B, L, D = 256, 128, 6
F_ATOM, F_BOND, FP = 39, 10, 128


def _lrelu(x):
    return jnp.where(x >= 0, x, 0.01 * x)


def _fused_kernel(atom_ref, bond_ref, aidx_ref, bidx_ref,
                  w_atom_ref, b_atom_ref, wnb_a_ref, wnb_b_ref, b_nb_ref,
                  w1_ref, w2_ref, b_align_ref, w_att_ref, b_att_ref,
                  w_ih_ref, b_ih_ref, w_hh_ref, b_hh_ref,
                  out_ref):
    atoms = atom_ref[0]
    bonds = bond_ref[0]
    aidx = aidx_ref[0]
    bidx = bidx_ref[0]

    dot = functools.partial(jnp.dot, preferred_element_type=jnp.float32)

    af = _lrelu(dot(atoms, w_atom_ref[...]) + b_atom_ref[...])

    align1 = jnp.sum(af * w1_ref[...], axis=1, keepdims=True) + b_align_ref[0, 0]

    iota = lax.broadcasted_iota(jnp.int32, (L, L), 1)
    w2 = w2_ref[...]
    b_nb = b_nb_ref[...]

    nfs = []
    scores = []
    valids = []
    for d in range(D):
        a_col = aidx[:, d:d + 1]
        b_col = bidx[:, d:d + 1]
        oh_a = (a_col == iota).astype(jnp.float32)
        oh_b = (b_col == iota).astype(jnp.float32)
        nbr_a = dot(oh_a, atoms)
        nbr_b = dot(oh_b, bonds)
        nf = _lrelu(dot(nbr_a, wnb_a_ref[...]) + dot(nbr_b, wnb_b_ref[...])
                    + b_nb)
        score = _lrelu(align1 + jnp.sum(nf * w2, axis=1, keepdims=True))
        valid = (a_col != L - 1)
        score = jnp.where(valid, score, score - 9e8)
        nfs.append(nf)
        scores.append(score)
        valids.append(valid.astype(jnp.float32))

    m = scores[0]
    for d in range(1, D):
        m = jnp.maximum(m, scores[d])
    exps = [jnp.exp(s - m) for s in scores]
    denom = exps[0]
    for d in range(1, D):
        denom = denom + exps[d]
    inv = 1.0 / denom

    acc = None
    tot = None
    for d in range(D):
        attn = exps[d] * inv * valids[d]
        term = attn * nfs[d]
        acc = term if acc is None else acc + term
        tot = attn if tot is None else tot + attn

    ctx_pre = dot(acc, w_att_ref[...]) + tot * b_att_ref[...]
    context = jnp.where(ctx_pre > 0, ctx_pre, jnp.expm1(ctx_pre))

    gi = dot(context, w_ih_ref[...]) + b_ih_ref[...]
    gh = dot(af, w_hh_ref[...]) + b_hh_ref[...]
    r = jax.nn.sigmoid(gi[:, :FP] + gh[:, :FP])
    z = jax.nn.sigmoid(gi[:, FP:2 * FP] + gh[:, FP:2 * FP])
    n = jnp.tanh(gi[:, 2 * FP:] + r * gh[:, 2 * FP:])
    hnew = (1.0 - z) * n + z * af
    out_ref[0] = jnp.maximum(hnew, 0.0)


def kernel(atom_list, bond_list, atom_degree_list, bond_degree_list, atom_mask,
           W_atom, b_atom, W_nb, b_nb, W_align, b_align, W_att, b_att,
           W_ih, W_hh, b_ih, b_hh):
    del atom_mask
    aidx = atom_degree_list.astype(jnp.int32)
    bidx = bond_degree_list.astype(jnp.int32)

    wnb_a = W_nb[:F_ATOM]
    wnb_b = W_nb[F_ATOM:]
    w1 = W_align[:FP, 0].reshape(1, FP)
    w2 = W_align[FP:, 0].reshape(1, FP)
    b_align2 = b_align.reshape(1, 1)
    w_ih_t = W_ih.T
    w_hh_t = W_hh.T

    rep2 = lambda arr: pl.BlockSpec(arr.shape, lambda i: (0,) * arr.ndim)
    row = lambda v: v.reshape(1, -1)

    grid = (B,)
    out = pl.pallas_call(
        _fused_kernel,
        grid=grid,
        in_specs=[
            pl.BlockSpec((1, L, F_ATOM), lambda i: (i, 0, 0)),
            pl.BlockSpec((1, L, F_BOND), lambda i: (i, 0, 0)),
            pl.BlockSpec((1, L, D), lambda i: (i, 0, 0)),
            pl.BlockSpec((1, L, D), lambda i: (i, 0, 0)),
            rep2(W_atom), rep2(row(b_atom)),
            rep2(wnb_a), rep2(wnb_b), rep2(row(b_nb)),
            rep2(w1), rep2(w2), rep2(b_align2),
            rep2(W_att), rep2(row(b_att)),
            rep2(w_ih_t), rep2(row(b_ih)),
            rep2(w_hh_t), rep2(row(b_hh)),
        ],
        out_specs=pl.BlockSpec((1, L, FP), lambda i: (i, 0, 0)),
        out_shape=jax.ShapeDtypeStruct((B, L, FP), jnp.float32),
    )(atom_list, bond_list, aidx, bidx,
      W_atom, row(b_atom), wnb_a, wnb_b, row(b_nb),
      w1, w2, b_align2, W_att, row(b_att),
      w_ih_t, row(b_ih), w_hh_t, row(b_hh))
    return out


# speedup vs baseline: 14.3848x; 14.3848x over previous
import functools

import jax
import jax.numpy as jnp
from jax import lax
from jax.experimental import pallas as pl

B, L, D = 256, 128, 6
F_ATOM, F_BOND, FP = 39, 10, 128


def _lrelu(x):
    return jnp.where(x >= 0, x, 0.01 * x)


def _fused_kernel(atom_ref, bond_ref, aidx_ref, bidx_ref,
                  w_atom_ref, b_atom_ref, wnb_a_ref, wnb_b_ref, b_nb_ref,
                  w1_ref, w2_ref, b_align_ref, w_att_ref, b_att_ref,
                  w_ih_ref, b_ih_ref, w_hh_ref, b_hh_ref,
                  out_ref):
    atoms = atom_ref[0]
    bonds = bond_ref[0]
    aidx = aidx_ref[0]
    bidx = bidx_ref[0]

    dot = functools.partial(jnp.dot, preferred_element_type=jnp.float32)

    af = _lrelu(dot(atoms, w_atom_ref[...]) + b_atom_ref[...])

    align1 = jnp.sum(af * w1_ref[...], axis=1, keepdims=True) + b_align_ref[0, 0]

    iota = lax.broadcasted_iota(jnp.int32, (L, L), 1)
    w2 = w2_ref[...]
    b_nb = b_nb_ref[...]

    nfs = []
    scores = []
    valids = []
    for d in range(D):
        a_col = aidx[:, d:d + 1]
        b_col = bidx[:, d:d + 1]
        oh_a = (a_col == iota).astype(jnp.float32)
        oh_b = (b_col == iota).astype(jnp.float32)
        nbr_a = dot(oh_a, atoms)
        nbr_b = dot(oh_b, bonds)
        nf = _lrelu(dot(nbr_a, wnb_a_ref[...]) + dot(nbr_b, wnb_b_ref[...])
                    + b_nb)
        score = _lrelu(align1 + jnp.sum(nf * w2, axis=1, keepdims=True))
        valid = (a_col != L - 1)
        score = jnp.where(valid, score, score - 9e8)
        nfs.append(nf)
        scores.append(score)
        valids.append(valid.astype(jnp.float32))

    m = scores[0]
    for d in range(1, D):
        m = jnp.maximum(m, scores[d])
    exps = [jnp.exp(s - m) for s in scores]
    denom = exps[0]
    for d in range(1, D):
        denom = denom + exps[d]
    inv = 1.0 / denom

    acc = None
    tot = None
    for d in range(D):
        attn = exps[d] * inv * valids[d]
        term = attn * nfs[d]
        acc = term if acc is None else acc + term
        tot = attn if tot is None else tot + attn

    ctx_pre = dot(acc, w_att_ref[...]) + tot * b_att_ref[...]
    context = jnp.where(ctx_pre > 0, ctx_pre, jnp.exp(ctx_pre) - 1.0)

    gi = dot(context, w_ih_ref[...]) + b_ih_ref[...]
    gh = dot(af, w_hh_ref[...]) + b_hh_ref[...]
    r = jax.nn.sigmoid(gi[:, :FP] + gh[:, :FP])
    z = jax.nn.sigmoid(gi[:, FP:2 * FP] + gh[:, FP:2 * FP])
    n = jnp.tanh(gi[:, 2 * FP:] + r * gh[:, 2 * FP:])
    hnew = (1.0 - z) * n + z * af
    out_ref[0] = jnp.maximum(hnew, 0.0)


def kernel(atom_list, bond_list, atom_degree_list, bond_degree_list, atom_mask,
           W_atom, b_atom, W_nb, b_nb, W_align, b_align, W_att, b_att,
           W_ih, W_hh, b_ih, b_hh):
    del atom_mask
    aidx = atom_degree_list.astype(jnp.int32)
    bidx = bond_degree_list.astype(jnp.int32)

    wnb_a = W_nb[:F_ATOM]
    wnb_b = W_nb[F_ATOM:]
    w1 = W_align[:FP, 0].reshape(1, FP)
    w2 = W_align[FP:, 0].reshape(1, FP)
    b_align2 = b_align.reshape(1, 1)
    w_ih_t = W_ih.T
    w_hh_t = W_hh.T

    rep2 = lambda arr: pl.BlockSpec(arr.shape, lambda i: (0,) * arr.ndim)
    row = lambda v: v.reshape(1, -1)

    grid = (B,)
    out = pl.pallas_call(
        _fused_kernel,
        grid=grid,
        in_specs=[
            pl.BlockSpec((1, L, F_ATOM), lambda i: (i, 0, 0)),
            pl.BlockSpec((1, L, F_BOND), lambda i: (i, 0, 0)),
            pl.BlockSpec((1, L, D), lambda i: (i, 0, 0)),
            pl.BlockSpec((1, L, D), lambda i: (i, 0, 0)),
            rep2(W_atom), rep2(row(b_atom)),
            rep2(wnb_a), rep2(wnb_b), rep2(row(b_nb)),
            rep2(w1), rep2(w2), rep2(b_align2),
            rep2(W_att), rep2(row(b_att)),
            rep2(w_ih_t), rep2(row(b_ih)),
            rep2(w_hh_t), rep2(row(b_hh)),
        ],
        out_specs=pl.BlockSpec((1, L, FP), lambda i: (i, 0, 0)),
        out_shape=jax.ShapeDtypeStruct((B, L, FP), jnp.float32),
    )(atom_list, bond_list, aidx, bidx,
      W_atom, row(b_atom), wnb_a, wnb_b, row(b_nb),
      w1, w2, b_align2, W_att, row(b_att),
      w_ih_t, row(b_ih), w_hh_t, row(b_hh))
    return out
